# Initial kernel scaffold; baseline (speedup 1.0000x reference)
import jax
import jax.numpy as jnp
from jax.experimental import pallas as pl


def kernel(input, weight):
    raise NotImplementedError("write your pallas kernel here")


# speedup vs baseline: 3.3573x; 3.3573x over previous
import functools

import jax
import jax.numpy as jnp
from jax import lax
from jax.experimental import pallas as pl
from jax.experimental.pallas import tpu as pltpu
from jax.experimental.pallas import tpu_sc as plsc

CH = 128


def _quant_block(w_ref, o_ref):
    w = w_ref[...]
    scale = jnp.maximum(jnp.max(jnp.abs(w), axis=1, keepdims=True) / 127.0, 1e-8)
    q = jnp.clip(jnp.round(w / scale), -127.0, 127.0) * scale
    o_ref[...] = q


def _quantize_table(weight):
    v, d = weight.shape
    rb = 2000
    assert v % rb == 0
    return pl.pallas_call(
        _quant_block,
        out_shape=jax.ShapeDtypeStruct((v, d), weight.dtype),
        grid=(v // rb,),
        in_specs=[pl.BlockSpec((rb, d), lambda i: (i, 0))],
        out_specs=pl.BlockSpec((rb, d), lambda i: (i, 0)),
    )(weight)


@functools.cache
def _make_gather(nw, nc, chunks, ch, d):
    b_per_w = chunks * ch
    mesh = plsc.VectorSubcoreMesh(core_axis_name="c", subcore_axis_name="s")

    @functools.partial(
        pl.kernel,
        out_type=jax.ShapeDtypeStruct((nw * b_per_w, d), jnp.float32),
        mesh=mesh,
        compiler_params=pltpu.CompilerParams(use_tc_tiling_on_sc=False),
        scratch_types=[
            pltpu.VMEM((chunks, ch), jnp.int32),
            pltpu.VMEM((ch, d), jnp.float32),
            pltpu.SemaphoreType.DMA,
        ],
    )
    def gather_k(idx_hbm, table_hbm, out_hbm, idx_v, rows_v, sem):
        wid = lax.axis_index("s") * nc + lax.axis_index("c")
        base = wid * b_per_w
        pltpu.sync_copy(idx_hbm.at[wid], idx_v)

        def body(j, carry):
            pltpu.async_copy(table_hbm.at[idx_v.at[j]], rows_v, sem).wait()
            pltpu.sync_copy(rows_v, out_hbm.at[pl.ds(base + j * ch, ch)])
            return carry

        lax.fori_loop(0, chunks, body, 0)

    return gather_k


def kernel(input, weight):
    v, d = weight.shape
    qw = _quantize_table(weight)

    idx = input.reshape(-1).astype(jnp.int32)
    b = idx.shape[0]
    info = plsc.get_sparse_core_info()
    nc, ns = info.num_cores, info.num_subcores
    nw = nc * ns
    grain = nw * CH
    b_pad = (b + grain - 1) // grain * grain
    if b_pad != b:
        idx = jnp.pad(idx, (0, b_pad - b))
    chunks = b_pad // grain
    idx3 = idx.reshape(nw, chunks, CH)

    out = _make_gather(nw, nc, chunks, CH, d)(idx3, qw)
    if b_pad != b:
        out = out[:b]
    return out.reshape(*input.shape, d)


# speedup vs baseline: 3.8393x; 1.1436x over previous
import functools

import jax
import jax.numpy as jnp
from jax import lax
from jax.experimental import pallas as pl
from jax.experimental.pallas import tpu as pltpu
from jax.experimental.pallas import tpu_sc as plsc

CH = 128


def _quant_block(w_ref, o_ref):
    w = w_ref[...]
    scale = jnp.maximum(jnp.max(jnp.abs(w), axis=1, keepdims=True) / 127.0, 1e-8)
    q = jnp.clip(jnp.round(w / scale), -127.0, 127.0) * scale
    o_ref[...] = q


def _quantize_table(weight):
    v, d = weight.shape
    rb = 2000
    assert v % rb == 0
    return pl.pallas_call(
        _quant_block,
        out_shape=jax.ShapeDtypeStruct((v, d), weight.dtype),
        grid=(v // rb,),
        in_specs=[pl.BlockSpec((rb, d), lambda i: (i, 0))],
        out_specs=pl.BlockSpec((rb, d), lambda i: (i, 0)),
    )(weight)


@functools.cache
def _make_gather(nw, nc, chunks, ch, d, k):
    b_per_w = chunks * ch
    assert chunks % (2 * k) == 0
    n_iter = chunks // (2 * k)
    mesh = plsc.VectorSubcoreMesh(core_axis_name="c", subcore_axis_name="s")

    @functools.partial(
        pl.kernel,
        out_type=jax.ShapeDtypeStruct((nw * b_per_w, d), jnp.float32),
        mesh=mesh,
        compiler_params=pltpu.CompilerParams(use_tc_tiling_on_sc=False),
        scratch_types=[
            pltpu.VMEM((chunks, ch), jnp.int32),
            pltpu.VMEM((k, ch, d), jnp.float32),
            pltpu.VMEM((k, ch, d), jnp.float32),
            pltpu.SemaphoreType.DMA,
            pltpu.SemaphoreType.DMA,
            pltpu.SemaphoreType.DMA,
            pltpu.SemaphoreType.DMA,
        ],
    )
    def gather_k(idx_hbm, table_hbm, out_hbm, idx_v, rows_a, rows_b,
                 sem_ga, sem_gb, sem_sa, sem_sb):
        wid = lax.axis_index("s") * nc + lax.axis_index("c")
        base = wid * b_per_w
        pltpu.sync_copy(idx_hbm.at[wid], idx_v)

        def gather_start(j, buf, sem):
            return pltpu.async_copy(table_hbm.at[idx_v.at[j]], buf, sem)

        def scatter_start(j, buf, sem):
            return pltpu.async_copy(buf, out_hbm.at[pl.ds(base + j * ch, ch)],
                                    sem)

        def drain(buf, sem):
            pltpu.make_async_copy(table_hbm.at[idx_v.at[0]], buf, sem).wait()

        for b in range(k):
            gather_start(b, rows_a.at[b], sem_ga)

        def body(t, carry):
            g0 = 2 * t
            c0 = g0 * k
            c1 = c0 + k
            for b in range(k):
                gather_start(c1 + b, rows_b.at[b], sem_gb)
            for b in range(k):
                drain(rows_a.at[b], sem_ga)
            for b in range(k):
                scatter_start(c0 + b, rows_a.at[b], sem_sa)
            for b in range(k):
                drain(rows_a.at[b], sem_sa)

            @pl.when(t + 1 < n_iter)
            def _():
                for b in range(k):
                    gather_start(c0 + 2 * k + b, rows_a.at[b], sem_ga)

            for b in range(k):
                drain(rows_b.at[b], sem_gb)
            for b in range(k):
                scatter_start(c1 + b, rows_b.at[b], sem_sb)
            for b in range(k):
                drain(rows_b.at[b], sem_sb)
            return carry

        lax.fori_loop(0, n_iter, body, 0)

    return gather_k


def kernel(input, weight):
    v, d = weight.shape
    qw = _quantize_table(weight)

    idx = input.reshape(-1).astype(jnp.int32)
    b = idx.shape[0]
    info = plsc.get_sparse_core_info()
    nc, ns = info.num_cores, info.num_subcores
    nw = nc * ns
    k = 4
    grain = nw * CH * 2 * k
    b_pad = (b + grain - 1) // grain * grain
    if b_pad != b:
        idx = jnp.pad(idx, (0, b_pad - b))
    chunks = b_pad // (nw * CH)
    idx3 = idx.reshape(nw, chunks, CH)

    out = _make_gather(nw, nc, chunks, CH, d, k)(idx3, qw)
    if b_pad != b:
        out = out[:b]
    return out.reshape(*input.shape, d)
